# Initial kernel scaffold; baseline (speedup 1.0000x reference)
import jax
import jax.numpy as jnp
from jax.experimental import pallas as pl


def kernel(x, conv_w, conv_b, gamma, beta):
    raise NotImplementedError("write your pallas kernel here")


# speedup vs baseline: 1.6256x; 1.6256x over previous
import jax
import jax.numpy as jnp
from jax.experimental import pallas as pl
from jax.experimental.pallas import tpu as pltpu

_BN_EPS = 1e-5


def _lane_roll(x, s):
    if s == 0:
        return x
    return jnp.concatenate([x[:, s:], x[:, :s]], axis=1)


def _make_conv_stats_kernel(B, M, Cout, shifts):
    def _body(x_ref, w_ref, mask_ref, y_ref, s1_ref, s2_ref):
        s1 = jnp.zeros((Cout, 1), jnp.float32)
        s2 = jnp.zeros((Cout, 1), jnp.float32)
        for b in range(B):
            xb = x_ref[b]
            acc = jnp.zeros((Cout, M), jnp.float32)
            for t in range(9):
                xs = _lane_roll(xb, shifts[t])
                if t != 4:
                    xs = xs * mask_ref[t:t + 1, :]
                acc = acc + jnp.dot(w_ref[t], xs,
                                    preferred_element_type=jnp.float32)
            y_ref[:, b * M:(b + 1) * M] = acc.astype(jnp.bfloat16)
            s1 = s1 + jnp.sum(acc, axis=1, keepdims=True)
            s2 = s2 + jnp.sum(acc * acc, axis=1, keepdims=True)
        s1_ref[...] = s1
        s2_ref[...] = s2

    return _body


def _make_bn_relu_kernel(B, M):
    def _body(y_ref, scale_ref, shift_ref, o_ref):
        o = jnp.maximum(y_ref[...].astype(jnp.float32) * scale_ref[...]
                        + shift_ref[...], 0.0)
        for b in range(B):
            o_ref[b] = o[:, b * M:(b + 1) * M]

    return _body


def kernel(x, conv_w, conv_b, gamma, beta):
    del conv_b
    N, Cin, H, W = x.shape
    Cout = conv_w.shape[0]
    M = H * W
    B = 8 if N % 8 == 0 else 1
    G = N // B

    shifts = tuple(di * W + dj for di in (-1, 0, 1) for dj in (-1, 0, 1))
    i = jnp.arange(M, dtype=jnp.int32) // W
    j = jnp.arange(M, dtype=jnp.int32) % W
    masks = jnp.stack([((i + di >= 0) & (i + di < H)
                        & (j + dj >= 0) & (j + dj < W)).astype(jnp.float32)
                       for di in (-1, 0, 1) for dj in (-1, 0, 1)], axis=0)

    w9 = conv_w.transpose(2, 3, 0, 1).reshape(9, Cout, Cin)
    x3 = x.reshape(N, Cin, M)

    y, s1, s2 = pl.pallas_call(
        _make_conv_stats_kernel(B, M, Cout, shifts),
        out_shape=(jax.ShapeDtypeStruct((Cout, N * M), jnp.bfloat16),
                   jax.ShapeDtypeStruct((G, Cout, 1), jnp.float32),
                   jax.ShapeDtypeStruct((G, Cout, 1), jnp.float32)),
        grid=(G,),
        in_specs=[
            pl.BlockSpec((B, Cin, M), lambda g: (g, 0, 0)),
            pl.BlockSpec((9, Cout, Cin), lambda g: (0, 0, 0)),
            pl.BlockSpec((9, M), lambda g: (0, 0)),
        ],
        out_specs=[
            pl.BlockSpec((Cout, B * M), lambda g: (0, g)),
            pl.BlockSpec((None, Cout, 1), lambda g: (g, 0, 0)),
            pl.BlockSpec((None, Cout, 1), lambda g: (g, 0, 0)),
        ],
        compiler_params=pltpu.CompilerParams(
            dimension_semantics=("parallel",)),
    )(x3, w9, masks)

    cnt = jnp.float32(N * M)
    mean = s1.sum(axis=0) / cnt
    var = s2.sum(axis=0) / cnt - mean * mean
    scale = gamma.reshape(Cout, 1) * jax.lax.rsqrt(var + _BN_EPS)
    shift = beta.reshape(Cout, 1) - mean * scale

    out = pl.pallas_call(
        _make_bn_relu_kernel(B, M),
        out_shape=jax.ShapeDtypeStruct((N, Cout, M), jnp.float32),
        grid=(G,),
        in_specs=[
            pl.BlockSpec((Cout, B * M), lambda g: (0, g)),
            pl.BlockSpec((Cout, 1), lambda g: (0, 0)),
            pl.BlockSpec((Cout, 1), lambda g: (0, 0)),
        ],
        out_specs=pl.BlockSpec((B, Cout, M), lambda g: (g, 0, 0)),
        compiler_params=pltpu.CompilerParams(
            dimension_semantics=("parallel",)),
    )(y, scale, shift)

    return out.reshape(N, Cout, H, W)
